# baseline (speedup 1.0000x reference)
import functools
import math

import jax
import jax.numpy as jnp
from jax import lax
from jax.experimental import pallas as pl
from jax.experimental.pallas import tpu as pltpu
from jax.experimental.pallas import tpu_sc as plsc

_NFEAT = 128
_NCLASS = 16
_B = 100000
_NC = 2
_NS = 16
_NW = _NC * _NS
_CHUNK = 400
_NSLOT = 2
_K = 8
_B_PAD = 102400
_SLICE = _B_PAD // _K
_CPW = _SLICE // (_NW * _CHUNK)
_RPW = _CPW * _CHUNK


def _sc_gather(x, idx_slice):
    mesh = plsc.VectorSubcoreMesh(core_axis_name="c", subcore_axis_name="s")

    @functools.partial(
        pl.kernel,
        out_type=jax.ShapeDtypeStruct((_SLICE, _NFEAT), jnp.float32),
        mesh=mesh,
        scratch_types=[
            pltpu.VMEM((_RPW,), jnp.int32),
            pltpu.VMEM((_NSLOT, _CHUNK, _NFEAT), jnp.float32),
            pltpu.SemaphoreType.DMA((_NSLOT,)),
            pltpu.SemaphoreType.DMA((_NSLOT,)),
        ],
    )
    def gather_kernel(x_hbm, idx_hbm, out_hbm, idx_v, rows_v, gsems, ssems):
        wid = lax.axis_index("s") * _NC + lax.axis_index("c")
        row0 = wid * _RPW
        pltpu.sync_copy(idx_hbm.at[pl.ds(row0, _RPW)], idx_v)

        def start_gather(c, slot):
            pltpu.async_copy(x_hbm.at[idx_v.at[pl.ds(c * _CHUNK, _CHUNK)]],
                             rows_v.at[slot], gsems.at[slot])

        def gather_desc(c, slot):
            return pltpu.make_async_copy(
                x_hbm.at[idx_v.at[pl.ds(c * _CHUNK, _CHUNK)]],
                rows_v.at[slot], gsems.at[slot])

        def scatter_desc(c, slot):
            return pltpu.make_async_copy(
                rows_v.at[slot],
                out_hbm.at[pl.ds(row0 + c * _CHUNK, _CHUNK)],
                ssems.at[slot])

        start_gather(0, 0)
        for j in range(_CPW):
            slot = j % _NSLOT
            gather_desc(j, slot).wait()
            scatter_desc(j, slot).start()
            n = j + 1
            if n < _CPW:
                if n >= _NSLOT:
                    scatter_desc(n - _NSLOT, n % _NSLOT).wait()
                start_gather(n, n % _NSLOT)
        for c in range(max(_CPW - _NSLOT, 0), _CPW):
            scatter_desc(c, c % _NSLOT).wait()

    return gather_kernel(x, idx_slice)


def _tc_fused(anchor, hop_feat, W1, b1, W2, b2, bm, blk0, n_rows):
    scale = 1.0 / math.sqrt(float(_NFEAT))

    def body(a_ref, h_ref, w1_ref, b1_ref, w2_ref, b2_ref, o_ref):
        a = a_ref[...]
        h0 = h_ref[0]
        h1 = h_ref[1]
        h2 = h_ref[2]
        ones = jnp.ones((_NFEAT, 1), jnp.float32)
        d0 = jnp.dot(a * a, ones, preferred_element_type=jnp.float32)
        d1 = jnp.dot(a * h0, ones, preferred_element_type=jnp.float32)
        d2 = jnp.dot(a * h1, ones, preferred_element_type=jnp.float32)
        d3 = jnp.dot(a * h2, ones, preferred_element_type=jnp.float32)
        r = 1.0 / jnp.maximum(jnp.sqrt(d0), 1e-12)
        l0 = d0 * r * r * scale
        rs = r * scale
        l1 = d1 * rs
        l2 = d2 * rs
        l3 = d3 * rs
        m = jnp.maximum(jnp.maximum(l0, l1), jnp.maximum(l2, l3))
        e0 = jnp.exp(l0 - m)
        e1 = jnp.exp(l1 - m)
        e2 = jnp.exp(l2 - m)
        e3 = jnp.exp(l3 - m)
        inv = 1.0 / (e0 + e1 + e2 + e3)
        pooled = ((e0 * r * inv) * a + (e1 * inv) * h0
                  + (e2 * inv) * h1 + (e3 * inv) * h2)
        h = jnp.dot(pooled, w1_ref[...], preferred_element_type=jnp.float32)
        h = jnp.maximum(h + b1_ref[...], 0.0)
        o = jnp.dot(h, w2_ref[...], preferred_element_type=jnp.float32) + b2_ref[...]
        om = jnp.max(o, axis=1, keepdims=True)
        o_ref[...] = (o - om) - jnp.log(
            jnp.sum(jnp.exp(o - om), axis=1, keepdims=True))

    return pl.pallas_call(
        body,
        grid=(n_rows // bm,),
        in_specs=[
            pl.BlockSpec((bm, _NFEAT), lambda i: (i, 0)),
            pl.BlockSpec((3, bm, _NFEAT), lambda i: (0, blk0 + i, 0)),
            pl.BlockSpec((_NFEAT, _NFEAT), lambda i: (0, 0)),
            pl.BlockSpec((1, _NFEAT), lambda i: (0, 0)),
            pl.BlockSpec((_NFEAT, _NCLASS), lambda i: (0, 0)),
            pl.BlockSpec((1, _NCLASS), lambda i: (0, 0)),
        ],
        out_specs=pl.BlockSpec((bm, _NCLASS), lambda i: (i, 0)),
        out_shape=jax.ShapeDtypeStruct((n_rows, _NCLASS), jnp.float32),
    )(anchor, hop_feat, W1, b1, W2, b2)


def kernel(x, hop_feat, idx, W1, b1, W2, b2):
    idx32 = idx.astype(jnp.int32)
    idx_pad = jnp.concatenate(
        [idx32, jnp.zeros((_B_PAD - _B,), jnp.int32)])
    b1r = b1.reshape(1, _NFEAT)
    b2r = b2.reshape(1, _NCLASS)
    bm = 800
    outs = []
    for k in range(_K):
        idx_slice = lax.slice_in_dim(idx_pad, k * _SLICE, (k + 1) * _SLICE)
        anchor_k = _sc_gather(x, idx_slice)
        n_rows = min(_B - k * _SLICE, _SLICE)
        outs.append(_tc_fused(anchor_k, hop_feat, W1, b1r, W2, b2r,
                              bm=bm, blk0=k * (_SLICE // bm), n_rows=n_rows))
    return jnp.concatenate(outs, axis=0)


# speedup vs baseline: 1.1310x; 1.1310x over previous
import functools
import math

import jax
import jax.numpy as jnp
from jax import lax
from jax.experimental import pallas as pl
from jax.experimental.pallas import tpu as pltpu
from jax.experimental.pallas import tpu_sc as plsc

_NFEAT = 128
_NCLASS = 16
_B = 100000
_NC = 2
_NS = 16
_NW = _NC * _NS
_CHUNK = 400
_NSLOT = 2
_K = 2
_B_PAD = 102400
_SLICE = _B_PAD // _K
_CPW = _SLICE // (_NW * _CHUNK)
_RPW = _CPW * _CHUNK


def _sc_gather(x, idx_slice):
    mesh = plsc.VectorSubcoreMesh(core_axis_name="c", subcore_axis_name="s")

    @functools.partial(
        pl.kernel,
        out_type=jax.ShapeDtypeStruct((_SLICE, _NFEAT), jnp.float32),
        mesh=mesh,
        scratch_types=[
            pltpu.VMEM((_RPW,), jnp.int32),
            pltpu.VMEM((_NSLOT, _CHUNK, _NFEAT), jnp.float32),
            pltpu.SemaphoreType.DMA((_NSLOT,)),
            pltpu.SemaphoreType.DMA((_NSLOT,)),
        ],
    )
    def gather_kernel(x_hbm, idx_hbm, out_hbm, idx_v, rows_v, gsems, ssems):
        wid = lax.axis_index("s") * _NC + lax.axis_index("c")
        row0 = wid * _RPW
        pltpu.sync_copy(idx_hbm.at[pl.ds(row0, _RPW)], idx_v)

        def start_gather(c, slot):
            pltpu.async_copy(x_hbm.at[idx_v.at[pl.ds(c * _CHUNK, _CHUNK)]],
                             rows_v.at[slot], gsems.at[slot])

        def gather_desc(c, slot):
            return pltpu.make_async_copy(
                x_hbm.at[idx_v.at[pl.ds(c * _CHUNK, _CHUNK)]],
                rows_v.at[slot], gsems.at[slot])

        def scatter_desc(c, slot):
            return pltpu.make_async_copy(
                rows_v.at[slot],
                out_hbm.at[pl.ds(row0 + c * _CHUNK, _CHUNK)],
                ssems.at[slot])

        start_gather(0, 0)
        for j in range(_CPW):
            slot = j % _NSLOT
            gather_desc(j, slot).wait()
            scatter_desc(j, slot).start()
            n = j + 1
            if n < _CPW:
                if n >= _NSLOT:
                    scatter_desc(n - _NSLOT, n % _NSLOT).wait()
                start_gather(n, n % _NSLOT)
        for c in range(max(_CPW - _NSLOT, 0), _CPW):
            scatter_desc(c, c % _NSLOT).wait()

    return gather_kernel(x, idx_slice)


def _tc_fused(anchor, hop_feat, W1, b1, W2, b2, bm, blk0, n_rows):
    scale = 1.0 / math.sqrt(float(_NFEAT))

    def body(a_ref, h_ref, w1_ref, b1_ref, w2_ref, b2_ref, o_ref):
        a = a_ref[...]
        h0 = h_ref[0]
        h1 = h_ref[1]
        h2 = h_ref[2]
        ones = jnp.ones((_NFEAT, 1), jnp.float32)
        d0 = jnp.dot(a * a, ones, preferred_element_type=jnp.float32)
        d1 = jnp.dot(a * h0, ones, preferred_element_type=jnp.float32)
        d2 = jnp.dot(a * h1, ones, preferred_element_type=jnp.float32)
        d3 = jnp.dot(a * h2, ones, preferred_element_type=jnp.float32)
        r = 1.0 / jnp.maximum(jnp.sqrt(d0), 1e-12)
        l0 = d0 * r * r * scale
        rs = r * scale
        l1 = d1 * rs
        l2 = d2 * rs
        l3 = d3 * rs
        m = jnp.maximum(jnp.maximum(l0, l1), jnp.maximum(l2, l3))
        e0 = jnp.exp(l0 - m)
        e1 = jnp.exp(l1 - m)
        e2 = jnp.exp(l2 - m)
        e3 = jnp.exp(l3 - m)
        inv = 1.0 / (e0 + e1 + e2 + e3)
        pooled = ((e0 * r * inv) * a + (e1 * inv) * h0
                  + (e2 * inv) * h1 + (e3 * inv) * h2)
        h = jnp.dot(pooled, w1_ref[...], preferred_element_type=jnp.float32)
        h = jnp.maximum(h + b1_ref[...], 0.0)
        o = jnp.dot(h, w2_ref[...], preferred_element_type=jnp.float32) + b2_ref[...]
        om = jnp.max(o, axis=1, keepdims=True)
        o_ref[...] = (o - om) - jnp.log(
            jnp.sum(jnp.exp(o - om), axis=1, keepdims=True))

    return pl.pallas_call(
        body,
        grid=(n_rows // bm,),
        in_specs=[
            pl.BlockSpec((bm, _NFEAT), lambda i: (i, 0)),
            pl.BlockSpec((3, bm, _NFEAT), lambda i: (0, blk0 + i, 0)),
            pl.BlockSpec((_NFEAT, _NFEAT), lambda i: (0, 0)),
            pl.BlockSpec((1, _NFEAT), lambda i: (0, 0)),
            pl.BlockSpec((_NFEAT, _NCLASS), lambda i: (0, 0)),
            pl.BlockSpec((1, _NCLASS), lambda i: (0, 0)),
        ],
        out_specs=pl.BlockSpec((bm, _NCLASS), lambda i: (i, 0)),
        out_shape=jax.ShapeDtypeStruct((n_rows, _NCLASS), jnp.float32),
    )(anchor, hop_feat, W1, b1, W2, b2)


def kernel(x, hop_feat, idx, W1, b1, W2, b2):
    idx32 = idx.astype(jnp.int32)
    idx_pad = jnp.concatenate(
        [idx32, jnp.zeros((_B_PAD - _B,), jnp.int32)])
    b1r = b1.reshape(1, _NFEAT)
    b2r = b2.reshape(1, _NCLASS)
    bm = 800
    outs = []
    for k in range(_K):
        idx_slice = lax.slice_in_dim(idx_pad, k * _SLICE, (k + 1) * _SLICE)
        anchor_k = _sc_gather(x, idx_slice)
        n_rows = min(_B - k * _SLICE, _SLICE)
        outs.append(_tc_fused(anchor_k, hop_feat, W1, b1r, W2, b2r,
                              bm=bm, blk0=k * (_SLICE // bm), n_rows=n_rows))
    return jnp.concatenate(outs, axis=0)


# speedup vs baseline: 1.3619x; 1.2042x over previous
import functools
import math

import jax
import jax.numpy as jnp
from jax import lax
from jax.experimental import pallas as pl
from jax.experimental.pallas import tpu as pltpu
from jax.experimental.pallas import tpu_sc as plsc

_NFEAT = 128
_NCLASS = 16
_B = 100000
_NC = 2
_NS = 16
_NW = _NC * _NS
_CHUNK = 400
_NSLOT = 2
_K = 4
_B_PAD = 102400
_SLICE = _B_PAD // _K
_CPW = _SLICE // (_NW * _CHUNK)
_RPW = _CPW * _CHUNK


def _sc_gather(x, idx_slice):
    mesh = plsc.VectorSubcoreMesh(core_axis_name="c", subcore_axis_name="s")

    @functools.partial(
        pl.kernel,
        out_type=jax.ShapeDtypeStruct((_SLICE, _NFEAT), jnp.float32),
        mesh=mesh,
        scratch_types=[
            pltpu.VMEM((_RPW,), jnp.int32),
            pltpu.VMEM((_NSLOT, _CHUNK, _NFEAT), jnp.float32),
            pltpu.SemaphoreType.DMA((_NSLOT,)),
            pltpu.SemaphoreType.DMA((_NSLOT,)),
        ],
    )
    def gather_kernel(x_hbm, idx_hbm, out_hbm, idx_v, rows_v, gsems, ssems):
        wid = lax.axis_index("s") * _NC + lax.axis_index("c")
        row0 = wid * _RPW
        pltpu.sync_copy(idx_hbm.at[pl.ds(row0, _RPW)], idx_v)

        def start_gather(c, slot):
            pltpu.async_copy(x_hbm.at[idx_v.at[pl.ds(c * _CHUNK, _CHUNK)]],
                             rows_v.at[slot], gsems.at[slot])

        def gather_desc(c, slot):
            return pltpu.make_async_copy(
                x_hbm.at[idx_v.at[pl.ds(c * _CHUNK, _CHUNK)]],
                rows_v.at[slot], gsems.at[slot])

        def scatter_desc(c, slot):
            return pltpu.make_async_copy(
                rows_v.at[slot],
                out_hbm.at[pl.ds(row0 + c * _CHUNK, _CHUNK)],
                ssems.at[slot])

        start_gather(0, 0)
        for j in range(_CPW):
            slot = j % _NSLOT
            gather_desc(j, slot).wait()
            scatter_desc(j, slot).start()
            n = j + 1
            if n < _CPW:
                if n >= _NSLOT:
                    scatter_desc(n - _NSLOT, n % _NSLOT).wait()
                start_gather(n, n % _NSLOT)
        for c in range(max(_CPW - _NSLOT, 0), _CPW):
            scatter_desc(c, c % _NSLOT).wait()

    return gather_kernel(x, idx_slice)


def _tc_fused(anchor, hop_feat, W1, b1, W2, b2, bm, blk0, n_rows):
    scale = 1.0 / math.sqrt(float(_NFEAT))

    def body(a_ref, h_ref, w1_ref, b1_ref, w2_ref, b2_ref, o_ref):
        a = a_ref[...]
        h0 = h_ref[0]
        h1 = h_ref[1]
        h2 = h_ref[2]
        ones = jnp.ones((_NFEAT, 1), jnp.float32)
        d0 = jnp.dot(a * a, ones, preferred_element_type=jnp.float32)
        d1 = jnp.dot(a * h0, ones, preferred_element_type=jnp.float32)
        d2 = jnp.dot(a * h1, ones, preferred_element_type=jnp.float32)
        d3 = jnp.dot(a * h2, ones, preferred_element_type=jnp.float32)
        r = 1.0 / jnp.maximum(jnp.sqrt(d0), 1e-12)
        l0 = d0 * r * r * scale
        rs = r * scale
        l1 = d1 * rs
        l2 = d2 * rs
        l3 = d3 * rs
        m = jnp.maximum(jnp.maximum(l0, l1), jnp.maximum(l2, l3))
        e0 = jnp.exp(l0 - m)
        e1 = jnp.exp(l1 - m)
        e2 = jnp.exp(l2 - m)
        e3 = jnp.exp(l3 - m)
        inv = 1.0 / (e0 + e1 + e2 + e3)
        pooled = ((e0 * r * inv) * a + (e1 * inv) * h0
                  + (e2 * inv) * h1 + (e3 * inv) * h2)
        h = jnp.dot(pooled, w1_ref[...], preferred_element_type=jnp.float32)
        h = jnp.maximum(h + b1_ref[...], 0.0)
        o = jnp.dot(h, w2_ref[...], preferred_element_type=jnp.float32) + b2_ref[...]
        om = jnp.max(o, axis=1, keepdims=True)
        o_ref[...] = (o - om) - jnp.log(
            jnp.sum(jnp.exp(o - om), axis=1, keepdims=True))

    return pl.pallas_call(
        body,
        grid=(n_rows // bm,),
        in_specs=[
            pl.BlockSpec((bm, _NFEAT), lambda i: (i, 0)),
            pl.BlockSpec((3, bm, _NFEAT), lambda i: (0, blk0 + i, 0)),
            pl.BlockSpec((_NFEAT, _NFEAT), lambda i: (0, 0)),
            pl.BlockSpec((1, _NFEAT), lambda i: (0, 0)),
            pl.BlockSpec((_NFEAT, _NCLASS), lambda i: (0, 0)),
            pl.BlockSpec((1, _NCLASS), lambda i: (0, 0)),
        ],
        out_specs=pl.BlockSpec((bm, _NCLASS), lambda i: (i, 0)),
        out_shape=jax.ShapeDtypeStruct((n_rows, _NCLASS), jnp.float32),
    )(anchor, hop_feat, W1, b1, W2, b2)


def kernel(x, hop_feat, idx, W1, b1, W2, b2):
    idx32 = idx.astype(jnp.int32)
    idx_pad = jnp.concatenate(
        [idx32, jnp.arange(_B_PAD - _B, dtype=jnp.int32)])
    b1r = b1.reshape(1, _NFEAT)
    b2r = b2.reshape(1, _NCLASS)
    bm = 800
    outs = []
    for k in range(_K):
        idx_slice = lax.slice_in_dim(idx_pad, k * _SLICE, (k + 1) * _SLICE)
        anchor_k = _sc_gather(x, idx_slice)
        n_rows = min(_B - k * _SLICE, _SLICE)
        outs.append(_tc_fused(anchor_k, hop_feat, W1, b1r, W2, b2r,
                              bm=bm, blk0=k * (_SLICE // bm), n_rows=n_rows))
    return jnp.concatenate(outs, axis=0)


# speedup vs baseline: 1.3648x; 1.0021x over previous
import functools
import math

import jax
import jax.numpy as jnp
from jax import lax
from jax.experimental import pallas as pl
from jax.experimental.pallas import tpu as pltpu
from jax.experimental.pallas import tpu_sc as plsc

_NFEAT = 128
_NCLASS = 16
_B = 100000
_NC = 2
_NS = 16
_NW = _NC * _NS
_CHUNK = 400
_NSLOT = 2
_K = 4
_B_PAD = 102400
_SLICE = _B_PAD // _K
_CPW = _SLICE // (_NW * _CHUNK)
_RPW = _CPW * _CHUNK


def _sc_gather(x, idx_slice):
    mesh = plsc.VectorSubcoreMesh(core_axis_name="c", subcore_axis_name="s")

    @functools.partial(
        pl.kernel,
        out_type=jax.ShapeDtypeStruct((_SLICE, _NFEAT), jnp.float32),
        mesh=mesh,
        scratch_types=[
            pltpu.VMEM((_RPW,), jnp.int32),
            pltpu.VMEM((_NSLOT, _CHUNK, _NFEAT), jnp.float32),
            pltpu.SemaphoreType.DMA((_NSLOT,)),
            pltpu.SemaphoreType.DMA((_NSLOT,)),
        ],
    )
    def gather_kernel(x_hbm, idx_hbm, out_hbm, idx_v, rows_v, gsems, ssems):
        wid = lax.axis_index("s") * _NC + lax.axis_index("c")
        row0 = wid * _RPW
        pltpu.sync_copy(idx_hbm.at[pl.ds(row0, _RPW)], idx_v)

        def start_gather(c, slot):
            pltpu.async_copy(x_hbm.at[idx_v.at[pl.ds(c * _CHUNK, _CHUNK)]],
                             rows_v.at[slot], gsems.at[slot])

        def gather_desc(c, slot):
            return pltpu.make_async_copy(
                x_hbm.at[idx_v.at[pl.ds(c * _CHUNK, _CHUNK)]],
                rows_v.at[slot], gsems.at[slot])

        def scatter_desc(c, slot):
            return pltpu.make_async_copy(
                rows_v.at[slot],
                out_hbm.at[pl.ds(row0 + c * _CHUNK, _CHUNK)],
                ssems.at[slot])

        start_gather(0, 0)
        for j in range(_CPW):
            slot = j % _NSLOT
            gather_desc(j, slot).wait()
            scatter_desc(j, slot).start()
            n = j + 1
            if n < _CPW:
                if n >= _NSLOT:
                    scatter_desc(n - _NSLOT, n % _NSLOT).wait()
                start_gather(n, n % _NSLOT)
        for c in range(max(_CPW - _NSLOT, 0), _CPW):
            scatter_desc(c, c % _NSLOT).wait()

    return gather_kernel(x, idx_slice)


def _tc_fused(anchor, hop_feat, W1, b1, W2, b2, bm, blk0, n_rows):
    scale = 1.0 / math.sqrt(float(_NFEAT))

    def body(a_ref, h_ref, w1_ref, b1_ref, w2_ref, b2_ref, o_ref):
        a = a_ref[...]
        h0 = h_ref[0]
        h1 = h_ref[1]
        h2 = h_ref[2]
        ones = jnp.ones((_NFEAT, 1), jnp.float32)
        d0 = jnp.dot(a * a, ones, preferred_element_type=jnp.float32)
        d1 = jnp.dot(a * h0, ones, preferred_element_type=jnp.float32)
        d2 = jnp.dot(a * h1, ones, preferred_element_type=jnp.float32)
        d3 = jnp.dot(a * h2, ones, preferred_element_type=jnp.float32)
        r = 1.0 / jnp.maximum(jnp.sqrt(d0), 1e-12)
        l0 = d0 * r * r * scale
        rs = r * scale
        l1 = d1 * rs
        l2 = d2 * rs
        l3 = d3 * rs
        m = jnp.maximum(jnp.maximum(l0, l1), jnp.maximum(l2, l3))
        e0 = jnp.exp(l0 - m)
        e1 = jnp.exp(l1 - m)
        e2 = jnp.exp(l2 - m)
        e3 = jnp.exp(l3 - m)
        inv = 1.0 / (e0 + e1 + e2 + e3)
        pooled = ((e0 * r * inv) * a + (e1 * inv) * h0
                  + (e2 * inv) * h1 + (e3 * inv) * h2)
        h = jnp.dot(pooled, w1_ref[...], preferred_element_type=jnp.float32)
        h = jnp.maximum(h + b1_ref[...], 0.0)
        o = jnp.dot(h, w2_ref[...], preferred_element_type=jnp.float32) + b2_ref[...]
        om = jnp.max(o, axis=1, keepdims=True)
        o_ref[...] = (o - om) - jnp.log(
            jnp.sum(jnp.exp(o - om), axis=1, keepdims=True))

    return pl.pallas_call(
        body,
        grid=(n_rows // bm,),
        in_specs=[
            pl.BlockSpec((bm, _NFEAT), lambda i: (i, 0)),
            pl.BlockSpec((3, bm, _NFEAT), lambda i: (0, blk0 + i, 0)),
            pl.BlockSpec((_NFEAT, _NFEAT), lambda i: (0, 0)),
            pl.BlockSpec((1, _NFEAT), lambda i: (0, 0)),
            pl.BlockSpec((_NFEAT, _NCLASS), lambda i: (0, 0)),
            pl.BlockSpec((1, _NCLASS), lambda i: (0, 0)),
        ],
        out_specs=pl.BlockSpec((bm, _NCLASS), lambda i: (i, 0)),
        out_shape=jax.ShapeDtypeStruct((n_rows, _NCLASS), jnp.float32),
    )(anchor, hop_feat, W1, b1, W2, b2)


def kernel(x, hop_feat, idx, W1, b1, W2, b2):
    idx32 = idx.astype(jnp.int32)
    b1r = b1.reshape(1, _NFEAT)
    b2r = b2.reshape(1, _NCLASS)
    bm = 1000
    bs = _B // _K
    pad = jnp.arange(_SLICE - bs, dtype=jnp.int32)
    outs = []
    for k in range(_K):
        idx_slice = jnp.concatenate(
            [lax.slice_in_dim(idx32, k * bs, (k + 1) * bs), pad])
        anchor_k = _sc_gather(x, idx_slice)
        outs.append(_tc_fused(anchor_k, hop_feat, W1, b1r, W2, b2r,
                              bm=bm, blk0=k * (bs // bm), n_rows=bs))
    return jnp.concatenate(outs, axis=0)


# speedup vs baseline: 1.5887x; 1.1640x over previous
import functools
import math

import jax
import jax.numpy as jnp
from jax import lax
from jax.experimental import pallas as pl
from jax.experimental.pallas import tpu as pltpu
from jax.experimental.pallas import tpu_sc as plsc

_NFEAT = 128
_NCLASS = 16
_B = 100000
_NC = 2
_NS = 16
_NW = _NC * _NS
_CHUNK = 400
_NSLOT = 2
_K = 1
_B_PAD = 102400
_SLICE = _B_PAD // _K
_CPW = _SLICE // (_NW * _CHUNK)
_RPW = _CPW * _CHUNK


def _sc_gather(x, idx_slice):
    mesh = plsc.VectorSubcoreMesh(core_axis_name="c", subcore_axis_name="s")

    @functools.partial(
        pl.kernel,
        out_type=jax.ShapeDtypeStruct((_SLICE, _NFEAT), jnp.float32),
        mesh=mesh,
        scratch_types=[
            pltpu.VMEM((_RPW,), jnp.int32),
            pltpu.VMEM((_NSLOT, _CHUNK, _NFEAT), jnp.float32),
            pltpu.SemaphoreType.DMA((_NSLOT,)),
            pltpu.SemaphoreType.DMA((_NSLOT,)),
        ],
    )
    def gather_kernel(x_hbm, idx_hbm, out_hbm, idx_v, rows_v, gsems, ssems):
        wid = lax.axis_index("s") * _NC + lax.axis_index("c")
        row0 = wid * _RPW
        pltpu.sync_copy(idx_hbm.at[pl.ds(row0, _RPW)], idx_v)

        def start_gather(c, slot):
            pltpu.async_copy(x_hbm.at[idx_v.at[pl.ds(c * _CHUNK, _CHUNK)]],
                             rows_v.at[slot], gsems.at[slot])

        def gather_desc(c, slot):
            return pltpu.make_async_copy(
                x_hbm.at[idx_v.at[pl.ds(c * _CHUNK, _CHUNK)]],
                rows_v.at[slot], gsems.at[slot])

        def scatter_desc(c, slot):
            return pltpu.make_async_copy(
                rows_v.at[slot],
                out_hbm.at[pl.ds(row0 + c * _CHUNK, _CHUNK)],
                ssems.at[slot])

        start_gather(0, 0)
        for j in range(_CPW):
            slot = j % _NSLOT
            gather_desc(j, slot).wait()
            scatter_desc(j, slot).start()
            n = j + 1
            if n < _CPW:
                if n >= _NSLOT:
                    scatter_desc(n - _NSLOT, n % _NSLOT).wait()
                start_gather(n, n % _NSLOT)
        for c in range(max(_CPW - _NSLOT, 0), _CPW):
            scatter_desc(c, c % _NSLOT).wait()

    return gather_kernel(x, idx_slice)


def _tc_fused(anchor, hop_feat, W1, b1, W2, b2, bm, blk0, n_rows):
    scale = 1.0 / math.sqrt(float(_NFEAT))

    def body(a_ref, h_ref, w1_ref, b1_ref, w2_ref, b2_ref, o_ref):
        a = a_ref[...]
        h0 = h_ref[0]
        h1 = h_ref[1]
        h2 = h_ref[2]
        ones = jnp.ones((_NFEAT, 1), jnp.float32)
        d0 = jnp.dot(a * a, ones, preferred_element_type=jnp.float32)
        d1 = jnp.dot(a * h0, ones, preferred_element_type=jnp.float32)
        d2 = jnp.dot(a * h1, ones, preferred_element_type=jnp.float32)
        d3 = jnp.dot(a * h2, ones, preferred_element_type=jnp.float32)
        r = 1.0 / jnp.maximum(jnp.sqrt(d0), 1e-12)
        l0 = d0 * r * r * scale
        rs = r * scale
        l1 = d1 * rs
        l2 = d2 * rs
        l3 = d3 * rs
        m = jnp.maximum(jnp.maximum(l0, l1), jnp.maximum(l2, l3))
        e0 = jnp.exp(l0 - m)
        e1 = jnp.exp(l1 - m)
        e2 = jnp.exp(l2 - m)
        e3 = jnp.exp(l3 - m)
        inv = 1.0 / (e0 + e1 + e2 + e3)
        pooled = ((e0 * r * inv) * a + (e1 * inv) * h0
                  + (e2 * inv) * h1 + (e3 * inv) * h2)
        h = jnp.dot(pooled, w1_ref[...], preferred_element_type=jnp.float32)
        h = jnp.maximum(h + b1_ref[...], 0.0)
        o = jnp.dot(h, w2_ref[...], preferred_element_type=jnp.float32) + b2_ref[...]
        om = jnp.max(o, axis=1, keepdims=True)
        o_ref[...] = (o - om) - jnp.log(
            jnp.sum(jnp.exp(o - om), axis=1, keepdims=True))

    return pl.pallas_call(
        body,
        grid=(n_rows // bm,),
        in_specs=[
            pl.BlockSpec((bm, _NFEAT), lambda i: (i, 0)),
            pl.BlockSpec((3, bm, _NFEAT), lambda i: (0, blk0 + i, 0)),
            pl.BlockSpec((_NFEAT, _NFEAT), lambda i: (0, 0)),
            pl.BlockSpec((1, _NFEAT), lambda i: (0, 0)),
            pl.BlockSpec((_NFEAT, _NCLASS), lambda i: (0, 0)),
            pl.BlockSpec((1, _NCLASS), lambda i: (0, 0)),
        ],
        out_specs=pl.BlockSpec((bm, _NCLASS), lambda i: (i, 0)),
        out_shape=jax.ShapeDtypeStruct((n_rows, _NCLASS), jnp.float32),
    )(anchor, hop_feat, W1, b1, W2, b2)


def kernel(x, hop_feat, idx, W1, b1, W2, b2):
    idx32 = idx.astype(jnp.int32)
    b1r = b1.reshape(1, _NFEAT)
    b2r = b2.reshape(1, _NCLASS)
    bm = 2000 if _K == 1 else 1000
    bs = _B // _K
    pad = jnp.arange(_SLICE - bs, dtype=jnp.int32)
    outs = []
    for k in range(_K):
        idx_slice = jnp.concatenate(
            [lax.slice_in_dim(idx32, k * bs, (k + 1) * bs), pad])
        anchor_k = _sc_gather(x, idx_slice)
        outs.append(_tc_fused(anchor_k, hop_feat, W1, b1r, W2, b2r,
                              bm=bm, blk0=k * (bs // bm), n_rows=bs))
    return jnp.concatenate(outs, axis=0)
